# Initial kernel scaffold; baseline (speedup 1.0000x reference)
import jax
import jax.numpy as jnp
from jax.experimental import pallas as pl


def kernel(context_vector, embeddings, thetas):
    raise NotImplementedError("write your pallas kernel here")


# speedup vs baseline: 1.4316x; 1.4316x over previous
import dataclasses
import functools

import jax
import jax.numpy as jnp
from jax import lax
from jax.experimental import pallas as pl
from jax.experimental.pallas import tpu as pltpu
from jax.experimental.pallas import tpu_sc as plsc

VOCAB = 100000
EMBED_DIM = 128
DEPTH = 17
N_INTERNAL = 2 ** DEPTH - 1
BATCH = 4096
CTX = 8

NC = 2
NS = 16
NW = NC * NS
BPW = BATCH // NW
NCHUNK = BPW * CTX // 128
NG = BPW // 16


def _sum8(vs):
    a0 = vs[0] + vs[1]
    a1 = vs[2] + vs[3]
    a2 = vs[4] + vs[5]
    a3 = vs[6] + vs[7]
    return (a0 + a1) + (a2 + a3)


_mesh = plsc.VectorSubcoreMesh(core_axis_name="c", subcore_axis_name="s")

_cp = pltpu.CompilerParams()
if "needs_layout_passes" in pltpu.CompilerParams.__dataclass_fields__:
    _cp = dataclasses.replace(_cp, needs_layout_passes=False)


@functools.partial(
    pl.kernel,
    out_type=[
        jax.ShapeDtypeStruct((NW, DEPTH, BPW), jnp.float32),
        jax.ShapeDtypeStruct((NW, BPW), jnp.int32),
    ],
    mesh=_mesh,
    compiler_params=_cp,
    scratch_types=[
        pltpu.VMEM((NCHUNK, 128), jnp.int32),
        pltpu.VMEM((128, EMBED_DIM), jnp.float32),
        pltpu.VMEM((128, EMBED_DIM), jnp.float32),
        pltpu.VMEM((EMBED_DIM, BPW), jnp.float32),
        pltpu.VMEM((BPW, EMBED_DIM), jnp.float32),
        pltpu.VMEM((BPW,), jnp.int32),
        pltpu.VMEM((DEPTH, BPW), jnp.float32),
        pltpu.VMEM((BPW,), jnp.int32),
        pltpu.SemaphoreType.DMA,
        pltpu.SemaphoreType.DMA,
    ],
)
def _hs_kernel(ctx_hbm, emb_hbm, th_hbm, scores_out, leaf_out,
               idx_v, ebuf0, ebuf1, xw_t, th_v, node_v, scores_v, leaf_v,
               sem0, sem1):
    wid = lax.axis_index("s") * NC + lax.axis_index("c")
    lane = jnp.arange(16, dtype=jnp.int32)

    pltpu.sync_copy(ctx_hbm.at[wid], idx_v)

    ebufs = [ebuf0, ebuf1]
    sems = [sem0, sem1]
    handles = [None, None]
    handles[0] = pltpu.async_copy(emb_hbm.at[idx_v.at[0]], ebuf0, sem0)
    for c in range(NCHUNK):
        pc = c % 2
        if c + 1 < NCHUNK:
            handles[1 - pc] = pltpu.async_copy(
                emb_hbm.at[idx_v.at[c + 1]], ebufs[1 - pc], sems[1 - pc])
        handles[pc].wait()
        buf = ebufs[pc]

        @pl.loop(0, 16)
        def _(b, c=c, buf=buf):
            r0 = b * 8
            bb = c * 16 + b
            bvec = jnp.full((16,), bb, dtype=jnp.int32)
            for dv in range(8):
                sl = pl.ds(dv * 16, 16)
                s = _sum8([buf[r0 + k, sl] for k in range(8)])
                plsc.store_scatter(xw_t, [dv * 16 + lane, bvec], s)

    @pl.loop(0, NG)
    def _(g):
        node_v[pl.ds(g * 16, 16)] = jnp.zeros((16,), jnp.int32)

    @pl.loop(0, DEPTH)
    def _(t):
        pltpu.async_copy(th_hbm.at[node_v], th_v, sem0).wait()

        @pl.loop(0, NG)
        def _(g):
            g16 = g * 16 + lane

            def dv_body(j, acc):
                for k in range(8):
                    dv = j * 8 + k
                    dvec = jnp.full((16,), dv, dtype=jnp.int32)
                    thv = plsc.load_gather(th_v, [g16, dvec])
                    xwv = xw_t[dv, pl.ds(g * 16, 16)]
                    acc = acc + thv * xwv
                return acc

            score = lax.fori_loop(0, EMBED_DIM // 8, dv_body,
                                  jnp.zeros((16,), jnp.float32))
            sl = pl.ds(g * 16, 16)
            scores_v[t, sl] = score
            nd = node_v[sl]
            node_v[sl] = nd * 2 + jnp.where(score < 0.0, 1, 2)

    @pl.loop(0, NG)
    def _(g):
        sl = pl.ds(g * 16, 16)
        leaf_v[sl] = node_v[sl] - N_INTERNAL

    pltpu.sync_copy(scores_v, scores_out.at[wid])
    pltpu.sync_copy(leaf_v, leaf_out.at[wid])


@jax.jit
def kernel(context_vector, embeddings, thetas):
    ctx3 = context_vector.astype(jnp.int32).reshape(NW, NCHUNK, 128)
    scores_t, leaf = _hs_kernel(ctx3, embeddings, thetas)
    scores = scores_t.transpose(0, 2, 1).reshape(BATCH, DEPTH)
    leaf_ix = leaf.reshape(BATCH)
    return leaf_ix, scores
